# Initial kernel scaffold; baseline (speedup 1.0000x reference)
import jax
import jax.numpy as jnp
from jax.experimental import pallas as pl


def kernel(sampled_points, smpl_vertices, tri_feats):
    raise NotImplementedError("write your pallas kernel here")


# speedup vs baseline: 1.1550x; 1.1550x over previous
from functools import partial

import jax
import jax.numpy as jnp
from jax.experimental import pallas as pl
from jax.experimental.pallas import tpu as pltpu
from jax.experimental.pallas import tpu_sc as plsc

N_RAYS = 1024
N_SAMPLES = 64
N_VERTS = 6890
FEAT_DIM = 96
KP = 6912
BR = 4
BQ = BR * N_SAMPLES
NB = (N_RAYS * N_SAMPLES) // BQ
PAD_KSQ = 1e30


def _argmin_body(pts_ref, vth_ref, vtl_ref, ksq_ref, idx_ref):
    q = pts_ref[...]
    qx, qy, qz = q[:, 0:1], q[:, 1:2], q[:, 2:3]
    qsq = (qx * qx + qy * qy) + qz * qz
    qp = jnp.concatenate([q, jnp.zeros((BQ, 125), jnp.float32)], axis=1)
    dn = (((1,), (0,)), ((), ()))
    dot = (
        jax.lax.dot_general(qp, vth_ref[...], dimension_numbers=dn,
                            preferred_element_type=jnp.float32)
        + jax.lax.dot_general(qp, vtl_ref[...], dimension_numbers=dn,
                              preferred_element_type=jnp.float32)
    )
    d2 = (qsq - 2.0 * dot) + ksq_ref[...]
    d = jnp.maximum(d2, 0.0)
    rowmin = jnp.min(d, axis=1, keepdims=True)
    iota = jax.lax.broadcasted_iota(jnp.int32, (BQ, KP), 1)
    cand = jnp.where(d == rowmin, iota, KP)
    idx_ref[0, 0, :] = jnp.min(cand, axis=1)


def _nearest_idx(points_flat, smpl_vertices):
    key_sq = jnp.sum(smpl_vertices * smpl_vertices, axis=1)
    vt = jnp.pad(smpl_vertices, ((0, KP - N_VERTS), (0, 125))).T
    vth = vt.astype(jnp.bfloat16).astype(jnp.float32)
    vtl = (vt - vth).astype(jnp.bfloat16).astype(jnp.float32)
    ksq = jnp.pad(key_sq, (0, KP - N_VERTS), constant_values=PAD_KSQ)
    ksq = ksq.reshape(1, KP)
    idx3 = pl.pallas_call(
        _argmin_body,
        grid=(NB,),
        in_specs=[
            pl.BlockSpec((BQ, 3), lambda i: (i, 0)),
            pl.BlockSpec((128, KP), lambda i: (0, 0)),
            pl.BlockSpec((128, KP), lambda i: (0, 0)),
            pl.BlockSpec((1, KP), lambda i: (0, 0)),
        ],
        out_specs=pl.BlockSpec((1, 1, BQ), lambda i: (i, 0, 0)),
        out_shape=jax.ShapeDtypeStruct((NB, 1, BQ), jnp.int32),
    )(points_flat, vth, vtl, ksq)
    return idx3.reshape(-1)


FEAT_PAD = 128


def _gather_sc(tri_feats_padded, idx):
    n = idx.shape[0]
    window = 128
    i2 = idx.reshape(1, n)
    mesh = plsc.VectorSubcoreMesh(core_axis_name="core", subcore_axis_name="subcore")

    @partial(
        pl.kernel,
        out_type=jax.ShapeDtypeStruct((n, FEAT_PAD), tri_feats_padded.dtype),
        mesh=mesh,
    )
    def k(x_hbm, i_hbm, o_hbm):
        def body(i_vmem, o_vmem):
            pltpu.sync_copy(x_hbm.at[i_vmem.at[0]], o_vmem)

        pltpu.emit_pipeline(
            body,
            grid=(n // window,),
            in_specs=[pl.BlockSpec((1, window), index_map=lambda i: (0, i))],
            out_specs=[pl.BlockSpec((window, FEAT_PAD), index_map=lambda i: (i, 0))],
            core_axis_name=("core", "subcore"),
            dimension_semantics=(pltpu.PARALLEL,),
        )(i_hbm, o_hbm)

    return k(tri_feats_padded, i2)


def kernel(sampled_points, smpl_vertices, tri_feats):
    sp_min = jnp.min(sampled_points, axis=1, keepdims=True)
    sp_max = jnp.max(sampled_points, axis=1, keepdims=True)
    sp_norm = (sampled_points - sp_min) / (sp_max - sp_min)
    points_flat = sp_norm.reshape(-1, 3)
    idx = _nearest_idx(points_flat, smpl_vertices)
    tf_pad = jnp.pad(tri_feats, ((0, 0), (0, FEAT_PAD - FEAT_DIM)))
    feats = _gather_sc(tf_pad, idx)[:, :FEAT_DIM]
    return feats, idx
